# baseline (speedup 1.0000x reference)
import jax
import jax.numpy as jnp
from jax.experimental import pallas as pl
from jax.experimental.pallas import tpu as pltpu

_DOT_DIMS = (((1,), (0,)), ((), ()))
_CB = 64
_HT = 8


def _head_kernel(x_ref, wc_ref, bc_ref, wr_ref, br_ref, wd_ref, bd_ref,
                 cls_ref, reg_ref, dir_ref):
    kstep = pl.program_id(1)
    H = x_ref.shape[2]
    wc = wc_ref[0]
    wr = wr_ref[0]
    wd = wd_ref[0]
    bc = bc_ref[...][:, :, None]
    br = br_ref[...][:, :, None]
    bd = bd_ref[...][:, :, None]

    def tile_body(t, carry):
        sl = pl.ds(t * _HT, _HT)
        xt = x_ref[0, :, sl, :]
        rows = [xt[:, i, :] for i in range(_HT)]

        def head(w):
            return jnp.stack(
                [jax.lax.dot_general(w, r, _DOT_DIMS,
                                     preferred_element_type=jnp.float32)
                 for r in rows], axis=1)

        pc = head(wc)
        pr = head(wr)
        pd = head(wd)

        @pl.when(kstep == 0)
        def _():
            cls_ref[0, :, sl, :] = pc + bc
            reg_ref[0, :, sl, :] = pr + br
            dir_ref[0, :, sl, :] = pd + bd

        @pl.when(kstep != 0)
        def _():
            cls_ref[0, :, sl, :] += pc
            reg_ref[0, :, sl, :] += pr
            dir_ref[0, :, sl, :] += pd

        return carry

    jax.lax.fori_loop(0, H // _HT, tile_body, 0)


def kernel(x, W_cls, b_cls, W_reg, b_reg, W_dir, b_dir):
    B, C, H, W = x.shape
    K = C // _CB
    oc, og, od = W_cls.shape[0], W_reg.shape[0], W_dir.shape[0]
    bc = b_cls.reshape(oc, 1)
    bg = b_reg.reshape(og, 1)
    bd = b_dir.reshape(od, 1)

    def wchunk(w, o):
        return jnp.transpose(w.reshape(o, K, _CB), (1, 0, 2))

    def wspec(o):
        return pl.BlockSpec((1, o, _CB), lambda b, k: (k, 0, 0))

    def bspec(o):
        return pl.BlockSpec((o, 1), lambda b, k: (0, 0))

    def ospec(o):
        return pl.BlockSpec((1, o, H, W), lambda b, k: (b, 0, 0, 0))

    outs = pl.pallas_call(
        _head_kernel,
        grid=(B, K),
        in_specs=[
            pl.BlockSpec((1, _CB, H, W), lambda b, k: (b, k, 0, 0)),
            wspec(oc), bspec(oc), wspec(og), bspec(og), wspec(od), bspec(od),
        ],
        out_specs=[ospec(oc), ospec(og), ospec(od)],
        out_shape=[
            jax.ShapeDtypeStruct((B, oc, H, W), x.dtype),
            jax.ShapeDtypeStruct((B, og, H, W), x.dtype),
            jax.ShapeDtypeStruct((B, od, H, W), x.dtype),
        ],
        compiler_params=pltpu.CompilerParams(
            dimension_semantics=("arbitrary", "arbitrary")),
    )(x, wchunk(W_cls, oc), bc, wchunk(W_reg, og), bg, wchunk(W_dir, od), bd)
    return outs


# speedup vs baseline: 1.4121x; 1.4121x over previous
import jax
import jax.numpy as jnp
from jax.experimental import pallas as pl
from jax.experimental.pallas import tpu as pltpu

_DOT_DIMS = (((1,), (0,)), ((), ()))
_HB = 40


def _head_kernel(x_ref, wc_ref, bc_ref, wr_ref, br_ref, wd_ref, bd_ref,
                 cls_ref, reg_ref, dir_ref):
    wc = wc_ref[...]
    wr = wr_ref[...]
    wd = wd_ref[...]
    bc = bc_ref[...]
    br = br_ref[...]
    bd = bd_ref[...]
    for h in range(_HB):
        xb = x_ref[0, :, h, :]
        cls_ref[0, :, h, :] = jax.lax.dot_general(
            wc, xb, _DOT_DIMS, preferred_element_type=jnp.float32) + bc
        reg_ref[0, :, h, :] = jax.lax.dot_general(
            wr, xb, _DOT_DIMS, preferred_element_type=jnp.float32) + br
        dir_ref[0, :, h, :] = jax.lax.dot_general(
            wd, xb, _DOT_DIMS, preferred_element_type=jnp.float32) + bd


def kernel(x, W_cls, b_cls, W_reg, b_reg, W_dir, b_dir):
    B, C, H, W = x.shape
    G = pl.cdiv(H, _HB)
    oc, og, od = W_cls.shape[0], W_reg.shape[0], W_dir.shape[0]
    bc = b_cls.reshape(oc, 1)
    bg = b_reg.reshape(og, 1)
    bd = b_dir.reshape(od, 1)

    def wspec(o):
        return pl.BlockSpec((o, C), lambda b, j: (0, 0))

    def bspec(o):
        return pl.BlockSpec((o, 1), lambda b, j: (0, 0))

    def ospec(o):
        return pl.BlockSpec((1, o, _HB, W), lambda b, j: (b, 0, j, 0))

    outs = pl.pallas_call(
        _head_kernel,
        grid=(B, G),
        in_specs=[
            pl.BlockSpec((1, C, _HB, W), lambda b, j: (b, 0, j, 0)),
            wspec(oc), bspec(oc), wspec(og), bspec(og), wspec(od), bspec(od),
        ],
        out_specs=[ospec(oc), ospec(og), ospec(od)],
        out_shape=[
            jax.ShapeDtypeStruct((B, oc, H, W), x.dtype),
            jax.ShapeDtypeStruct((B, og, H, W), x.dtype),
            jax.ShapeDtypeStruct((B, od, H, W), x.dtype),
        ],
        compiler_params=pltpu.CompilerParams(
            dimension_semantics=("parallel", "parallel")),
    )(x, W_cls, bc, W_reg, bg, W_dir, bd)
    return outs


# speedup vs baseline: 3.3575x; 2.3776x over previous
import jax
import jax.numpy as jnp
from jax.experimental import pallas as pl
from jax.experimental.pallas import tpu as pltpu

_PB = 6912


def _head_kernel(x_ref, w_ref, b_ref, cls_ref, reg_ref, dir_ref):
    res = jax.lax.dot_general(
        w_ref[...], x_ref[0], (((1,), (1,)), ((), ())),
        preferred_element_type=jnp.float32) + b_ref[...]
    cls_ref[0] = res[0:2]
    reg_ref[0] = res[2:16]
    dir_ref[0] = res[16:20]


def kernel(x, W_cls, b_cls, W_reg, b_reg, W_dir, b_dir):
    B, C, H, W = x.shape
    HW = H * W
    G = pl.cdiv(HW, _PB)
    oc, og, od = W_cls.shape[0], W_reg.shape[0], W_dir.shape[0]
    xt = jnp.transpose(x, (0, 2, 3, 1)).reshape(B, HW, C)
    wall = jnp.concatenate([W_cls, W_reg, W_dir], axis=0)
    ball = jnp.concatenate([b_cls, b_reg, b_dir]).reshape(-1, 1)
    no = wall.shape[0]

    def ospec(o):
        return pl.BlockSpec((1, o, _PB), lambda b, j: (b, 0, j))

    outs = pl.pallas_call(
        _head_kernel,
        grid=(B, G),
        in_specs=[
            pl.BlockSpec((1, _PB, C), lambda b, j: (b, j, 0)),
            pl.BlockSpec((no, C), lambda b, j: (0, 0)),
            pl.BlockSpec((no, 1), lambda b, j: (0, 0)),
        ],
        out_specs=[ospec(oc), ospec(og), ospec(od)],
        out_shape=[
            jax.ShapeDtypeStruct((B, oc, HW), x.dtype),
            jax.ShapeDtypeStruct((B, og, HW), x.dtype),
            jax.ShapeDtypeStruct((B, od, HW), x.dtype),
        ],
        compiler_params=pltpu.CompilerParams(
            dimension_semantics=("parallel", "parallel")),
    )(xt, wall, ball)
    cls_o, reg_o, dir_o = outs
    return (cls_o.reshape(B, oc, H, W),
            reg_o.reshape(B, og, H, W),
            dir_o.reshape(B, od, H, W))


# speedup vs baseline: 4.2829x; 1.2756x over previous
import jax
import jax.numpy as jnp
from jax.experimental import pallas as pl
from jax.experimental.pallas import tpu as pltpu

_DOT_DIMS = (((1,), (0,)), ((), ()))
_WB = 8


def _head_kernel(x_ref, w_ref, b_ref, cls_ref, reg_ref, dir_ref):
    wt = w_ref[...]
    bias = b_ref[...]
    for w in range(_WB):
        xw = x_ref[0, :, w, :]
        r = jax.lax.dot_general(
            xw, wt, _DOT_DIMS, preferred_element_type=jnp.float32) + bias
        rt = r.T
        cls_ref[0, :, w, :] = rt[0:2]
        reg_ref[0, :, w, :] = rt[2:16]
        dir_ref[0, :, w, :] = rt[16:20]


def kernel(x, W_cls, b_cls, W_reg, b_reg, W_dir, b_dir):
    B, C, H, W = x.shape
    G = W // _WB
    oc, og, od = W_cls.shape[0], W_reg.shape[0], W_dir.shape[0]
    xt = jnp.transpose(x, (0, 2, 3, 1))
    wall = jnp.concatenate([W_cls, W_reg, W_dir], axis=0).T
    ball = jnp.concatenate([b_cls, b_reg, b_dir]).reshape(1, -1)
    no = wall.shape[1]

    def ospec(o):
        return pl.BlockSpec((1, o, _WB, H), lambda b, j: (b, 0, j, 0))

    outs = pl.pallas_call(
        _head_kernel,
        grid=(B, G),
        in_specs=[
            pl.BlockSpec((1, H, _WB, C), lambda b, j: (b, 0, j, 0)),
            pl.BlockSpec((C, no), lambda b, j: (0, 0)),
            pl.BlockSpec((1, no), lambda b, j: (0, 0)),
        ],
        out_specs=[ospec(oc), ospec(og), ospec(od)],
        out_shape=[
            jax.ShapeDtypeStruct((B, oc, W, H), x.dtype),
            jax.ShapeDtypeStruct((B, og, W, H), x.dtype),
            jax.ShapeDtypeStruct((B, od, W, H), x.dtype),
        ],
        compiler_params=pltpu.CompilerParams(
            dimension_semantics=("parallel", "parallel")),
    )(xt, wall, ball)
    return tuple(o.transpose(0, 1, 3, 2) for o in outs)


# speedup vs baseline: 5.5796x; 1.3028x over previous
import jax
import jax.numpy as jnp
from jax.experimental import pallas as pl
from jax.experimental.pallas import tpu as pltpu

_DOT_DIMS = (((1,), (0,)), ((), ()))
_WB = 24


def _head_kernel(x_ref, w_ref, b_ref, cls_ref, reg_ref, dir_ref):
    wt = w_ref[...]
    bias = b_ref[...]
    for w in range(_WB):
        xw = x_ref[0, :, w, :]
        r = jax.lax.dot_general(
            xw, wt, _DOT_DIMS, preferred_element_type=jnp.float32) + bias
        rt = r.T
        cls_ref[0, :, w, :] = rt[0:2]
        reg_ref[0, :, w, :] = rt[2:16]
        dir_ref[0, :, w, :] = rt[16:20]


def kernel(x, W_cls, b_cls, W_reg, b_reg, W_dir, b_dir):
    B, C, H, W = x.shape
    G = W // _WB
    oc, og, od = W_cls.shape[0], W_reg.shape[0], W_dir.shape[0]
    xt = jnp.transpose(x, (0, 2, 3, 1))
    wall = jnp.concatenate([W_cls, W_reg, W_dir], axis=0).T
    ball = jnp.concatenate([b_cls, b_reg, b_dir]).reshape(1, -1)
    no = wall.shape[1]

    def ospec(o):
        return pl.BlockSpec((1, o, _WB, H), lambda b, j: (b, 0, j, 0))

    outs = pl.pallas_call(
        _head_kernel,
        grid=(B, G),
        in_specs=[
            pl.BlockSpec((1, H, _WB, C), lambda b, j: (b, 0, j, 0)),
            pl.BlockSpec((C, no), lambda b, j: (0, 0)),
            pl.BlockSpec((1, no), lambda b, j: (0, 0)),
        ],
        out_specs=[ospec(oc), ospec(og), ospec(od)],
        out_shape=[
            jax.ShapeDtypeStruct((B, oc, W, H), x.dtype),
            jax.ShapeDtypeStruct((B, og, W, H), x.dtype),
            jax.ShapeDtypeStruct((B, od, W, H), x.dtype),
        ],
        compiler_params=pltpu.CompilerParams(
            dimension_semantics=("parallel", "parallel")),
    )(xt, wall, ball)
    return tuple(o.transpose(0, 1, 3, 2) for o in outs)
